# baseline (speedup 1.0000x reference)
import functools
import math

import jax
import jax.numpy as jnp
from jax import lax
from jax.experimental import pallas as pl
from jax.experimental.pallas import tpu as pltpu
from jax.experimental.pallas import tpu_sc as plsc

_L = 16
_C0 = 128
_NBUF = 4


def _emb_kernel(n_seq, t_len, d):
    info = plsc.get_sparse_core_info()
    nc, ns = info.num_cores, info.num_subcores
    nw = nc * ns
    s_per_w = n_seq // nw
    assert n_seq == nw * s_per_w and d % _L == 0
    assert _C0 <= t_len < 2 * _C0 and s_per_w >= 2 * _NBUF
    c1 = t_len - _C0
    scale = math.sqrt(d)

    mesh = plsc.VectorSubcoreMesh(core_axis_name="c", subcore_axis_name="s")

    @functools.partial(
        pl.kernel,
        out_type=jax.ShapeDtypeStruct((n_seq, t_len, d), jnp.float32),
        mesh=mesh,
        compiler_params=pltpu.CompilerParams(use_tc_tiling_on_sc=False),
        scratch_types=[
            pltpu.VMEM((s_per_w, t_len), jnp.int32),
            [pltpu.VMEM((t_len, d), jnp.float32) for _ in range(_NBUF)],
            [pltpu.VMEM((t_len, d), jnp.float32) for _ in range(_NBUF)],
            [pltpu.SemaphoreType.DMA for _ in range(_NBUF)],
            [pltpu.SemaphoreType.DMA for _ in range(_NBUF)],
        ],
    )
    def emb(x_hbm, table_hbm, out_hbm, idx_v, gbuf, sbuf, gsem, ssem):
        wid = lax.axis_index("s") * nc + lax.axis_index("c")
        seq0 = wid * s_per_w
        pltpu.sync_copy(x_hbm.at[pl.ds(seq0, s_per_w)], idx_v)

        def start_gather(j, b):
            pltpu.async_copy(
                table_hbm.at[idx_v.at[j, pl.ds(0, _C0)]],
                gbuf[b].at[pl.ds(0, _C0)], gsem[b])
            pltpu.async_copy(
                table_hbm.at[idx_v.at[j, pl.ds(_C0, c1)]],
                gbuf[b].at[pl.ds(_C0, c1)], gsem[b])

        def wait_gather(b):
            pltpu.make_async_copy(
                table_hbm.at[idx_v.at[0, pl.ds(0, _C0)]],
                gbuf[b].at[pl.ds(0, _C0)], gsem[b]).wait()
            pltpu.make_async_copy(
                table_hbm.at[idx_v.at[0, pl.ds(_C0, c1)]],
                gbuf[b].at[pl.ds(_C0, c1)], gsem[b]).wait()

        def start_scatter(j, b):
            pltpu.async_copy(sbuf[b], out_hbm.at[seq0 + j], ssem[b])

        def wait_scatter(b):
            pltpu.make_async_copy(sbuf[b], out_hbm.at[seq0], ssem[b]).wait()

        def do_scale(b):
            src, dst = gbuf[b], sbuf[b]

            @pl.loop(0, t_len)
            def _(r):
                for c in range(d // _L):
                    sl = pl.ds(c * _L, _L)
                    dst[r, sl] = src[r, sl] * scale

        for b in range(_NBUF):
            start_gather(b, b)
        for b in range(_NBUF):
            wait_gather(b)
            do_scale(b)
            start_scatter(b, b)
            start_gather(b + _NBUF, b)
        @pl.loop(_NBUF, s_per_w - _NBUF, step=_NBUF)
        def _(j0):
            for b in range(_NBUF):
                j = j0 + b
                wait_scatter(b)
                wait_gather(b)
                do_scale(b)
                start_scatter(j, b)
                start_gather(j + _NBUF, b)
        for b in range(_NBUF):
            j = s_per_w - _NBUF + b
            wait_scatter(b)
            wait_gather(b)
            do_scale(b)
            start_scatter(j, b)
        for b in range(_NBUF):
            wait_scatter(b)

    return emb


def kernel(x, table):
    n_seq, t_len = x.shape
    d = table.shape[1]
    return _emb_kernel(n_seq, t_len, d)(x.astype(jnp.int32), table)
